# baseline (speedup 1.0000x reference)
import jax
import jax.numpy as jnp
from jax import lax
from jax.experimental import pallas as pl
from jax.experimental.pallas import tpu as pltpu

_HB = 32


def _unpool_body(v_ref, x_ref, out_ref):
    x = x_ref[0]
    v = v_ref[0]
    vt = jnp.transpose(v, (0, 2, 1))
    vf = jnp.repeat(vt, 2, axis=2)

    hb, c, w = x.shape

    wi = lax.broadcasted_iota(jnp.int32, x.shape, 2)
    nb = jnp.where((wi & 1) == 0,
                   pltpu.roll(x, w - 1, axis=2),
                   pltpu.roll(x, 1, axis=2))
    wx = jnp.maximum(x, nb)

    wr = wx.reshape(hb // 2, 2, c, w)
    m = jnp.maximum(wr[:, 0], wr[:, 1])

    xr = x.reshape(hb // 2, 2, c, w)
    oe = jnp.where(xr[:, 0] == m, vf, 0.0)
    oo = jnp.where(xr[:, 1] == m, vf, 0.0)
    out_ref[0] = jnp.stack([oe, oo], axis=1).reshape(hb, c, w)


def kernel(input_tensor, pool_input):
    B, H, W, C = pool_input.shape
    Ho, Wo = H // 2, W // 2
    nh = H // _HB

    xt = jnp.transpose(pool_input, (0, 1, 3, 2))

    out = pl.pallas_call(
        _unpool_body,
        grid=(B, nh),
        in_specs=[
            pl.BlockSpec((1, _HB // 2, Wo, C), lambda b, h: (b, h, 0, 0)),
            pl.BlockSpec((1, _HB, C, W), lambda b, h: (b, h, 0, 0)),
        ],
        out_specs=pl.BlockSpec((1, _HB, C, W), lambda b, h: (b, h, 0, 0)),
        out_shape=jax.ShapeDtypeStruct((B, H, C, W), pool_input.dtype),
        compiler_params=pltpu.CompilerParams(
            dimension_semantics=("parallel", "arbitrary"),
        ),
    )(input_tensor, xt)
    return jnp.transpose(out, (0, 1, 3, 2))


# speedup vs baseline: 14.1128x; 14.1128x over previous
import jax
import jax.numpy as jnp
from jax import lax
from jax.experimental import pallas as pl
from jax.experimental.pallas import tpu as pltpu

_HB = 32


def _unpool_body(v_ref, x_ref, out_ref):
    x = x_ref[0]
    v = v_ref[0]
    nr, wo, nc = v.shape
    vt = jnp.transpose(v, (0, 2, 1))
    jj = lax.broadcasted_iota(jnp.int32, (wo, 2 * wo), 0)
    ww = lax.broadcasted_iota(jnp.int32, (wo, 2 * wo), 1)
    expand = (jj == ww // 2).astype(vt.dtype)
    vf = jax.lax.dot_general(
        vt.reshape(nr * nc, wo), expand,
        dimension_numbers=(((1,), (0,)), ((), ())),
        preferred_element_type=jnp.float32,
    ).reshape(nr, nc, 2 * wo)

    hb, c, w = x.shape

    wi = lax.broadcasted_iota(jnp.int32, x.shape, 2)
    nb = jnp.where((wi & 1) == 0,
                   pltpu.roll(x, w - 1, axis=2),
                   pltpu.roll(x, 1, axis=2))
    wx = jnp.maximum(x, nb)

    wr = wx.reshape(hb // 2, 2, c, w)
    m = jnp.maximum(wr[:, 0], wr[:, 1])

    xr = x.reshape(hb // 2, 2, c, w)
    oe = jnp.where(xr[:, 0] == m, vf, 0.0)
    oo = jnp.where(xr[:, 1] == m, vf, 0.0)
    out_ref[0] = jnp.stack([oe, oo], axis=1).reshape(hb, c, w)


def kernel(input_tensor, pool_input):
    B, H, W, C = pool_input.shape
    Ho, Wo = H // 2, W // 2
    nh = H // _HB

    xt = jnp.transpose(pool_input, (0, 1, 3, 2))

    out = pl.pallas_call(
        _unpool_body,
        grid=(B, nh),
        in_specs=[
            pl.BlockSpec((1, _HB // 2, Wo, C), lambda b, h: (b, h, 0, 0)),
            pl.BlockSpec((1, _HB, C, W), lambda b, h: (b, h, 0, 0)),
        ],
        out_specs=pl.BlockSpec((1, _HB, C, W), lambda b, h: (b, h, 0, 0)),
        out_shape=jax.ShapeDtypeStruct((B, H, C, W), pool_input.dtype),
        compiler_params=pltpu.CompilerParams(
            dimension_semantics=("parallel", "arbitrary"),
        ),
    )(input_tensor, xt)
    return jnp.transpose(out, (0, 1, 3, 2))


# speedup vs baseline: 14.5391x; 1.0302x over previous
import jax
import jax.numpy as jnp
from jax import lax
from jax.experimental import pallas as pl
from jax.experimental.pallas import tpu as pltpu

_HB = 32


def _unpool_body(v_ref, x_ref, out_ref):
    x = x_ref[0]
    v = v_ref[0]
    nr, wo, nc = v.shape
    vt = jnp.transpose(v, (0, 2, 1))
    jj = lax.broadcasted_iota(jnp.int32, (wo, 2 * wo), 0)
    ww = lax.broadcasted_iota(jnp.int32, (wo, 2 * wo), 1)
    expand = (jj == ww // 2).astype(vt.dtype)
    vf = jax.lax.dot_general(
        vt.reshape(nr * nc, wo), expand,
        dimension_numbers=(((1,), (0,)), ((), ())),
        preferred_element_type=jnp.float32,
        precision=lax.Precision.HIGHEST,
    ).reshape(nr, nc, 2 * wo)

    hb, c, w = x.shape

    y = jnp.maximum(x, pltpu.roll(x, 1, axis=2))
    yr = y.reshape(hb // 2, 2, c, w)
    m2 = jnp.maximum(yr[:, 0], yr[:, 1])
    wi = lax.broadcasted_iota(jnp.int32, m2.shape, 2)
    m = jnp.where((wi & 1) == 1, m2,
                  pltpu.roll(m2, w - 1, axis=2))

    xr = x.reshape(hb // 2, 2, c, w)
    oe = jnp.where(xr[:, 0] == m, vf, 0.0)
    oo = jnp.where(xr[:, 1] == m, vf, 0.0)
    out_ref[0] = jnp.stack([oe, oo], axis=1).reshape(hb, c, w)


def kernel(input_tensor, pool_input):
    B, H, W, C = pool_input.shape
    Ho, Wo = H // 2, W // 2
    nh = H // _HB

    xt = jnp.transpose(pool_input, (0, 1, 3, 2))

    out = pl.pallas_call(
        _unpool_body,
        grid=(B, nh),
        in_specs=[
            pl.BlockSpec((1, _HB // 2, Wo, C), lambda b, h: (b, h, 0, 0)),
            pl.BlockSpec((1, _HB, C, W), lambda b, h: (b, h, 0, 0)),
        ],
        out_specs=pl.BlockSpec((1, _HB, C, W), lambda b, h: (b, h, 0, 0)),
        out_shape=jax.ShapeDtypeStruct((B, H, C, W), pool_input.dtype),
        compiler_params=pltpu.CompilerParams(
            dimension_semantics=("parallel", "arbitrary"),
        ),
    )(input_tensor, xt)
    return jnp.transpose(out, (0, 1, 3, 2))


# speedup vs baseline: 18.4730x; 1.2706x over previous
import jax
import jax.numpy as jnp
from jax import lax
from jax.experimental import pallas as pl
from jax.experimental.pallas import tpu as pltpu

_HB = 64


def _unpool_body(v_ref, x_ref, out_ref):
    x = x_ref[0]
    v = v_ref[0]
    nr, wo, nc = v.shape
    vt = jnp.transpose(v, (0, 2, 1))
    jj = lax.broadcasted_iota(jnp.int32, (wo, 2 * wo), 0)
    ww = lax.broadcasted_iota(jnp.int32, (wo, 2 * wo), 1)
    expand = (jj == ww // 2).astype(vt.dtype)
    vf = jax.lax.dot_general(
        vt.reshape(nr * nc, wo), expand,
        dimension_numbers=(((1,), (0,)), ((), ())),
        preferred_element_type=jnp.float32,
        precision=lax.Precision.HIGHEST,
    ).reshape(nr, nc, 2 * wo)

    hb, c, w = x.shape

    y = jnp.maximum(x, pltpu.roll(x, 1, axis=2))
    yr = y.reshape(hb // 2, 2, c, w)
    m2 = jnp.maximum(yr[:, 0], yr[:, 1])
    wi = lax.broadcasted_iota(jnp.int32, m2.shape, 2)
    m = jnp.where((wi & 1) == 1, m2,
                  pltpu.roll(m2, w - 1, axis=2))

    xr = x.reshape(hb // 2, 2, c, w)
    oe = jnp.where(xr[:, 0] == m, vf, 0.0)
    oo = jnp.where(xr[:, 1] == m, vf, 0.0)
    out_ref[0] = jnp.stack([oe, oo], axis=1).reshape(hb, c, w)


def kernel(input_tensor, pool_input):
    B, H, W, C = pool_input.shape
    Ho, Wo = H // 2, W // 2
    nh = H // _HB

    xt = jnp.transpose(pool_input, (0, 1, 3, 2))

    out = pl.pallas_call(
        _unpool_body,
        grid=(B, nh),
        in_specs=[
            pl.BlockSpec((1, _HB // 2, Wo, C), lambda b, h: (b, h, 0, 0)),
            pl.BlockSpec((1, _HB, C, W), lambda b, h: (b, h, 0, 0)),
        ],
        out_specs=pl.BlockSpec((1, _HB, C, W), lambda b, h: (b, h, 0, 0)),
        out_shape=jax.ShapeDtypeStruct((B, H, C, W), pool_input.dtype),
        compiler_params=pltpu.CompilerParams(
            dimension_semantics=("parallel", "arbitrary"),
        ),
    )(input_tensor, xt)
    return jnp.transpose(out, (0, 1, 3, 2))


# speedup vs baseline: 21.2630x; 1.1510x over previous
import jax
import jax.numpy as jnp
from jax import lax
from jax.experimental import pallas as pl
from jax.experimental.pallas import tpu as pltpu

_HB = 128


def _unpool_body(v_ref, x_ref, out_ref):
    x = x_ref[0]
    v = v_ref[0]
    nr, wo, nc = v.shape
    vt = jnp.transpose(v, (0, 2, 1))
    jj = lax.broadcasted_iota(jnp.int32, (wo, 2 * wo), 0)
    ww = lax.broadcasted_iota(jnp.int32, (wo, 2 * wo), 1)
    expand = (jj == ww // 2).astype(vt.dtype)
    vf = jax.lax.dot_general(
        vt.reshape(nr * nc, wo), expand,
        dimension_numbers=(((1,), (0,)), ((), ())),
        preferred_element_type=jnp.float32,
        precision=lax.Precision.HIGHEST,
    ).reshape(nr, nc, 2 * wo)

    hb, c, w = x.shape

    y = jnp.maximum(x, pltpu.roll(x, 1, axis=2))
    yr = y.reshape(hb // 2, 2, c, w)
    m2 = jnp.maximum(yr[:, 0], yr[:, 1])
    wi = lax.broadcasted_iota(jnp.int32, m2.shape, 2)
    m = jnp.where((wi & 1) == 1, m2,
                  pltpu.roll(m2, w - 1, axis=2))

    xr = x.reshape(hb // 2, 2, c, w)
    oe = jnp.where(xr[:, 0] == m, vf, 0.0)
    oo = jnp.where(xr[:, 1] == m, vf, 0.0)
    out_ref[0] = jnp.stack([oe, oo], axis=1).reshape(hb, c, w)


def kernel(input_tensor, pool_input):
    B, H, W, C = pool_input.shape
    Ho, Wo = H // 2, W // 2
    nh = H // _HB

    xt = jnp.transpose(pool_input, (0, 1, 3, 2))

    out = pl.pallas_call(
        _unpool_body,
        grid=(B, nh),
        in_specs=[
            pl.BlockSpec((1, _HB // 2, Wo, C), lambda b, h: (b, h, 0, 0)),
            pl.BlockSpec((1, _HB, C, W), lambda b, h: (b, h, 0, 0)),
        ],
        out_specs=pl.BlockSpec((1, _HB, C, W), lambda b, h: (b, h, 0, 0)),
        out_shape=jax.ShapeDtypeStruct((B, H, C, W), pool_input.dtype),
        compiler_params=pltpu.CompilerParams(
            dimension_semantics=("parallel", "arbitrary"),
        ),
    )(input_tensor, xt)
    return jnp.transpose(out, (0, 1, 3, 2))
